# baseline (speedup 1.0000x reference)
import jax
import jax.numpy as jnp
from jax.experimental import pallas as pl
from jax.experimental.pallas import tpu as pltpu


def _body(idx_ref, tab_ref, b1_ref, b2_ref, w1_hbm, w2_hbm, out_hbm,
          w1_vmem, w2_vmem, res_vmem, sem1, sem2, sem_out):
    cp1 = pltpu.make_async_copy(w1_hbm, w1_vmem, sem1)
    cp2 = pltpu.make_async_copy(w2_hbm, w2_vmem, sem2)
    cp1.start()
    cp2.start()

    idx = idx_ref[:, :]
    cols = jax.lax.broadcasted_iota(
        jnp.int32, (idx.shape[0], tab_ref.shape[0]), 1)
    onehot = (idx == cols).astype(jnp.float32)
    prompt = jnp.dot(onehot, tab_ref[:, :], preferred_element_type=jnp.float32)

    cp1.wait()
    h = jnp.tanh(
        jnp.dot(prompt, w1_vmem[:, :], preferred_element_type=jnp.float32)
        + b1_ref[:, :]
    )

    cp2.wait()
    res_vmem[:, :] = (
        jnp.dot(h, w2_vmem[:, :], preferred_element_type=jnp.float32)
        + b2_ref[:, :]
    )

    B = out_hbm.shape[0]
    outs = [pltpu.make_async_copy(res_vmem, out_hbm.at[b], sem_out)
            for b in range(B)]
    for cp in outs:
        cp.start()
    for cp in outs:
        cp.wait()


def kernel(tokens, batch_size, pre_prompt, embd_table, W1, b1, W2, b2):
    B = tokens.shape[0]
    P = pre_prompt.shape[0]
    D, H = W1.shape
    return pl.pallas_call(
        _body,
        in_specs=[
            pl.BlockSpec((P, 1), lambda: (0, 0)),
            pl.BlockSpec((P, D), lambda: (0, 0)),
            pl.BlockSpec((1, H), lambda: (0, 0)),
            pl.BlockSpec((1, D), lambda: (0, 0)),
            pl.BlockSpec(memory_space=pltpu.MemorySpace.HBM),
            pl.BlockSpec(memory_space=pltpu.MemorySpace.HBM),
        ],
        out_specs=pl.BlockSpec(memory_space=pltpu.MemorySpace.HBM),
        out_shape=jax.ShapeDtypeStruct((B, P, D), jnp.float32),
        scratch_shapes=[
            pltpu.VMEM((D, H), jnp.float32),
            pltpu.VMEM((H, D), jnp.float32),
            pltpu.VMEM((P, D), jnp.float32),
            pltpu.SemaphoreType.DMA,
            pltpu.SemaphoreType.DMA,
            pltpu.SemaphoreType.DMA,
        ],
    )(
        pre_prompt.reshape(P, 1),
        embd_table,
        b1.reshape(1, H),
        b2.reshape(1, D),
        W1,
        W2,
    )


# speedup vs baseline: 1.3830x; 1.3830x over previous
import jax
import jax.numpy as jnp
from jax.experimental import pallas as pl
from jax.experimental.pallas import tpu as pltpu


def _body(idx_ref, tab_hbm, w1_hbm, b1_hbm, w2_hbm, b2_hbm, out_ref,
          tab_v, w1_v, b1_v, w2_v, b2_v, s_tab, s_w1, s_b1, s_w2, s_b2):
    cps = [
        pltpu.make_async_copy(tab_hbm, tab_v, s_tab),
        pltpu.make_async_copy(w1_hbm, w1_v, s_w1),
        pltpu.make_async_copy(b1_hbm, b1_v, s_b1),
        pltpu.make_async_copy(w2_hbm, w2_v, s_w2),
        pltpu.make_async_copy(b2_hbm, b2_v, s_b2),
    ]
    for cp in cps:
        cp.start()

    idx_row = idx_ref[:, :]
    n_rows = tab_v.shape[0]
    rows = jax.lax.broadcasted_iota(
        jnp.int32, (n_rows, idx_row.shape[1]), 0)
    onehot_t = (rows == idx_row).astype(jnp.float32)

    cps[0].wait()
    prompt = jax.lax.dot_general(
        onehot_t, tab_v[:, :], (((0,), (0,)), ((), ())),
        preferred_element_type=jnp.float32)

    cps[1].wait()
    cps[2].wait()
    h = jnp.tanh(
        jnp.dot(prompt, w1_v[:, :], preferred_element_type=jnp.float32)
        + b1_v[:, :]
    )

    cps[3].wait()
    cps[4].wait()
    out_ref[:, :] = (
        jnp.dot(h, w2_v[:, :], preferred_element_type=jnp.float32)
        + b2_v[:, :]
    )


def kernel(tokens, batch_size, pre_prompt, embd_table, W1, b1, W2, b2):
    B = tokens.shape[0]
    P = pre_prompt.shape[0]
    D, H = W1.shape
    N = embd_table.shape[0]
    hbm = pl.BlockSpec(memory_space=pltpu.MemorySpace.HBM)
    res = pl.pallas_call(
        _body,
        in_specs=[pl.BlockSpec((1, P), lambda: (0, 0)),
                  hbm, hbm, hbm, hbm, hbm],
        out_shape=jax.ShapeDtypeStruct((P, D), jnp.float32),
        scratch_shapes=[
            pltpu.VMEM((N, D), jnp.float32),
            pltpu.VMEM((D, H), jnp.float32),
            pltpu.VMEM((1, H), jnp.float32),
            pltpu.VMEM((H, D), jnp.float32),
            pltpu.VMEM((1, D), jnp.float32),
            pltpu.SemaphoreType.DMA,
            pltpu.SemaphoreType.DMA,
            pltpu.SemaphoreType.DMA,
            pltpu.SemaphoreType.DMA,
            pltpu.SemaphoreType.DMA,
        ],
    )(
        pre_prompt.reshape(1, P),
        embd_table,
        W1,
        b1.reshape(1, H),
        W2,
        b2.reshape(1, D),
    )
    return jnp.broadcast_to(res[None], (B, P, D))
